# baseline (device time: 16395 ns/iter reference)
import jax
import jax.numpy as jnp
from jax import lax
from jax.experimental import pallas as pl
from jax.experimental.pallas import tpu as pltpu

M = 1024
NOUT = 512
HALF = M // 2
KC = 8
CR = HALF // KC


def kernel(x):

    def body(x_ref, out_ref, xsrc, local_d, local_r, sendx, recvx, recvz,
             stage_sems, ldma_sems, sendx_sems, recvx_sems,
             sendz_sems, recvz_sems):
        my_x = lax.axis_index("x")
        my_y = lax.axis_index("y")
        my_z = lax.axis_index("z")
        s = lax.rem(my_z, 2)
        peer_x = (1 - my_x, my_y, my_z)
        peer_z = (my_x, my_y, my_z + 1 - 2 * s)
        col_me = my_x * NOUT
        col_peer = (1 - my_x) * NOUT

        stage_dmas = []
        for k in range(KC):
            dma = pltpu.make_async_copy(
                x_ref.at[0, pl.ds(s * HALF + k * CR, CR), pl.ds(col_peer, NOUT)],
                xsrc.at[pl.ds(k * CR, CR), :],
                stage_sems.at[k],
            )
            dma.start()
            stage_dmas.append(dma)
        ldma_d = pltpu.make_async_copy(
            x_ref.at[0, pl.ds(s * HALF, HALF), pl.ds(col_me, NOUT)],
            local_d, ldma_sems.at[0],
        )
        ldma_d.start()
        ldma_r = pltpu.make_async_copy(
            x_ref.at[0, pl.ds((1 - s) * HALF, HALF), pl.ds(col_me, NOUT)],
            local_r, ldma_sems.at[1],
        )
        ldma_r.start()

        barrier_sem = pltpu.get_barrier_semaphore()
        for nbr in (peer_x, peer_z):
            pl.semaphore_signal(
                barrier_sem, inc=1, device_id=nbr,
                device_id_type=pl.DeviceIdType.MESH,
            )
        pl.semaphore_wait(barrier_sem, 2)

        x_rdmas = []
        for k in range(KC):
            c = pl.ds(k * CR, CR)
            stage_dmas[k].wait()
            sendx[c, :] = xsrc[c, :].astype(jnp.bfloat16)
            rdma = pltpu.make_async_remote_copy(
                src_ref=sendx.at[c, :],
                dst_ref=recvx.at[c, :],
                send_sem=sendx_sems.at[k],
                recv_sem=recvx_sems.at[k],
                device_id=peer_x,
                device_id_type=pl.DeviceIdType.MESH,
            )
            rdma.start()
            x_rdmas.append(rdma)

        ldma_d.wait()
        z_rdmas = []
        for k in range(KC):
            c = pl.ds(k * CR, CR)
            x_rdmas[k].wait_recv()
            rdma = pltpu.make_async_remote_copy(
                src_ref=recvx.at[c, :],
                dst_ref=recvz.at[c, :],
                send_sem=sendz_sems.at[k],
                recv_sem=recvz_sems.at[k],
                device_id=peer_z,
                device_id_type=pl.DeviceIdType.MESH,
            )
            rdma.start()
            z_rdmas.append(rdma)
            out_ref[pl.ds(s * HALF + k * CR, CR), :] = (
                local_d[c, :].astype(jnp.bfloat16) + recvx[c, :]
            )

        ldma_r.wait()
        for k in range(KC):
            c = pl.ds(k * CR, CR)
            z_rdmas[k].wait_recv()
            out_ref[pl.ds((1 - s) * HALF + k * CR, CR), :] = (
                local_r[c, :].astype(jnp.bfloat16) + recvz[c, :]
            )

        for k in range(KC):
            x_rdmas[k].wait_send()
            z_rdmas[k].wait_send()

    return pl.pallas_call(
        body,
        out_shape=jax.ShapeDtypeStruct((M, NOUT), jnp.bfloat16),
        in_specs=[pl.BlockSpec(memory_space=pl.ANY)],
        out_specs=pl.BlockSpec(memory_space=pltpu.VMEM),
        scratch_shapes=[
            pltpu.VMEM((HALF, NOUT), jnp.float32),
            pltpu.VMEM((HALF, NOUT), jnp.float32),
            pltpu.VMEM((HALF, NOUT), jnp.float32),
            pltpu.VMEM((HALF, NOUT), jnp.bfloat16),
            pltpu.VMEM((HALF, NOUT), jnp.bfloat16),
            pltpu.VMEM((HALF, NOUT), jnp.bfloat16),
            pltpu.SemaphoreType.DMA((KC,)),
            pltpu.SemaphoreType.DMA((2,)),
            pltpu.SemaphoreType.DMA((KC,)),
            pltpu.SemaphoreType.DMA((KC,)),
            pltpu.SemaphoreType.DMA((KC,)),
            pltpu.SemaphoreType.DMA((KC,)),
        ],
        compiler_params=pltpu.CompilerParams(collective_id=0),
    )(x)


# device time: 15661 ns/iter; 1.0469x vs baseline; 1.0469x over previous
import jax
import jax.numpy as jnp
from jax import lax
from jax.experimental import pallas as pl
from jax.experimental.pallas import tpu as pltpu

M = 1024
NOUT = 512
HALF = M // 2
KC = 16
CR = HALF // KC


def kernel(x):

    def body(x_ref, out_ref, sendx, recvx, recvz,
             sendx_sems, recvx_sems, sendz_sems, recvz_sems):
        my_x = lax.axis_index("x")
        my_y = lax.axis_index("y")
        my_z = lax.axis_index("z")
        s = lax.rem(my_z, 2)
        peer_x = (1 - my_x, my_y, my_z)
        peer_z = (my_x, my_y, my_z + 1 - 2 * s)

        barrier_sem = pltpu.get_barrier_semaphore()
        for nbr in (peer_x, peer_z):
            pl.semaphore_signal(
                barrier_sem, inc=1, device_id=nbr,
                device_id_type=pl.DeviceIdType.MESH,
            )
        pl.semaphore_wait(barrier_sem, 2)

        x_rdmas = []
        for k in range(KC):
            c = pl.ds(k * CR, CR)
            row = pl.ds(s * HALF + k * CR, CR)

            @pl.when(my_x == 0)
            def _(c=c, row=row):
                sendx[c, :] = x_ref[0, row, NOUT:].astype(jnp.bfloat16)

            @pl.when(my_x == 1)
            def _(c=c, row=row):
                sendx[c, :] = x_ref[0, row, :NOUT].astype(jnp.bfloat16)

            rdma = pltpu.make_async_remote_copy(
                src_ref=sendx.at[c, :],
                dst_ref=recvx.at[c, :],
                send_sem=sendx_sems.at[k],
                recv_sem=recvx_sems.at[k],
                device_id=peer_x,
                device_id_type=pl.DeviceIdType.MESH,
            )
            rdma.start()
            x_rdmas.append(rdma)

        z_rdmas = []
        for k in range(KC):
            c = pl.ds(k * CR, CR)
            x_rdmas[k].wait_recv()
            rdma = pltpu.make_async_remote_copy(
                src_ref=recvx.at[c, :],
                dst_ref=recvz.at[c, :],
                send_sem=sendz_sems.at[k],
                recv_sem=recvz_sems.at[k],
                device_id=peer_z,
                device_id_type=pl.DeviceIdType.MESH,
            )
            rdma.start()
            z_rdmas.append(rdma)

            row = pl.ds(s * HALF + k * CR, CR)

            @pl.when(my_x == 0)
            def _(c=c, row=row):
                out_ref[row, :] = (
                    x_ref[0, row, :NOUT].astype(jnp.bfloat16) + recvx[c, :]
                )

            @pl.when(my_x == 1)
            def _(c=c, row=row):
                out_ref[row, :] = (
                    x_ref[0, row, NOUT:].astype(jnp.bfloat16) + recvx[c, :]
                )

        for k in range(KC):
            c = pl.ds(k * CR, CR)
            z_rdmas[k].wait_recv()
            row = pl.ds((1 - s) * HALF + k * CR, CR)

            @pl.when(my_x == 0)
            def _(c=c, row=row):
                out_ref[row, :] = (
                    x_ref[0, row, :NOUT].astype(jnp.bfloat16) + recvz[c, :]
                )

            @pl.when(my_x == 1)
            def _(c=c, row=row):
                out_ref[row, :] = (
                    x_ref[0, row, NOUT:].astype(jnp.bfloat16) + recvz[c, :]
                )

        for k in range(KC):
            x_rdmas[k].wait_send()
            z_rdmas[k].wait_send()

    return pl.pallas_call(
        body,
        out_shape=jax.ShapeDtypeStruct((M, NOUT), jnp.bfloat16),
        in_specs=[pl.BlockSpec(memory_space=pltpu.VMEM)],
        out_specs=pl.BlockSpec(memory_space=pltpu.VMEM),
        scratch_shapes=[
            pltpu.VMEM((HALF, NOUT), jnp.bfloat16),
            pltpu.VMEM((HALF, NOUT), jnp.bfloat16),
            pltpu.VMEM((HALF, NOUT), jnp.bfloat16),
            pltpu.SemaphoreType.DMA((KC,)),
            pltpu.SemaphoreType.DMA((KC,)),
            pltpu.SemaphoreType.DMA((KC,)),
            pltpu.SemaphoreType.DMA((KC,)),
        ],
        compiler_params=pltpu.CompilerParams(collective_id=0),
    )(x)
